# baseline (speedup 1.0000x reference)
import jax
import jax.numpy as jnp
from jax import lax
from jax.experimental import pallas as pl
from jax.experimental.pallas import tpu as pltpu
from jax.experimental.pallas import tpu_sc as plsc

NPX = 100000
NCH = 64
E = 1600000
K = 10
N_ITERS = 5

NC = 2
NT = 16
NW = NC * NT
SLICE = 3200
NPXP = NW * SLICE
HALF = NPXP // NC
VECS = SLICE // 16
EPT = E // NW
C = 2000
NCHUNK = EPT // C
CV = C // 16
TCBLK = 1024
NSLOT = 4
MAIN = (NCHUNK // (2 * NSLOT)) * 2 * NSLOT
REM = NCHUNK - MAIN
assert 0 < REM <= NSLOT and MAIN >= 2 * NSLOT and C % 16 == 0
assert (MAIN // NSLOT) % 2 == 0


def _yproj_body(s_ref, d_ref, tau_ref, o_ref):
    s = s_ref[...]
    d = d_ref[...]
    sd = jnp.dot(s, d, preferred_element_type=jnp.float32)
    o_ref[...] = tau_ref[...] * jnp.sum(d * sd, axis=0, keepdims=True)


def _yproj(S, Dp, taup):
    return pl.pallas_call(
        _yproj_body,
        grid=(NPXP // TCBLK,),
        in_specs=[
            pl.BlockSpec((NCH, NCH), lambda i: (0, 0)),
            pl.BlockSpec((NCH, TCBLK), lambda i: (0, i)),
            pl.BlockSpec((1, TCBLK), lambda i: (0, i)),
        ],
        out_specs=pl.BlockSpec((1, TCBLK), lambda i: (0, i)),
        out_shape=jax.ShapeDtypeStruct((1, NPXP), jnp.float32),
    )(S, Dp, taup)


def _sc_body(src_hbm, dst_hbm, w_hbm, yp_hbm, mu_hbm,
             xout_hbm, part_hbm, xsh_hbm,
             xs_sh, acc_sh,
             t0_v, t1_v, out_v, yp_v, dinv_v, tmp_v, tmp2_v, ts_v, zer_v,
             src0, src1, src2, src3,
             w0, w1, w2, w3,
             xg0, xg1, xg2, xg3,
             da0, da1, da2, da3,
             db0, db1, db2, db3,
             mu_v, semL, semG, semS, gsem):
    cid = lax.axis_index("c")
    sid = lax.axis_index("s")
    wid = cid * NT + sid
    gown = cid * HALF + sid * SLICE
    fsl = (1 - cid) * HALF + sid * SLICE
    ebase = wid * EPT
    psl = pl.ds(gown, SLICE)
    xsl = pl.ds(fsl, SLICE)
    srcb = (src0, src1, src2, src3)
    wb = (w0, w1, w2, w3)
    xgb = (xg0, xg1, xg2, xg3)
    dstb = ((da0, da1, da2, da3), (db0, db1, db2, db3))

    def cross_barrier():
        plsc.subcore_barrier()

        @pl.when(sid == 0)
        def _():
            pl.semaphore_signal(gsem, 1, device_id={"c": 1 - cid, "s": 0})
            pl.semaphore_wait(gsem, 1)
        plsc.subcore_barrier()

    def issue_l(c, s, bank):
        eb = ebase + c * C
        pltpu.async_copy(src_hbm.at[pl.ds(eb, C)], srcb[s], semL.at[s])
        pltpu.async_copy(w_hbm.at[pl.ds(eb, C)], wb[s], semL.at[s])
        pltpu.async_copy(dst_hbm.at[pl.ds(eb, C)], dstb[bank][s], semL.at[s])

    def drain_l(s, bank):
        pltpu.make_async_copy(src_hbm.at[pl.ds(0, C)], srcb[s],
                              semL.at[s]).wait()
        pltpu.make_async_copy(w_hbm.at[pl.ds(0, C)], wb[s],
                              semL.at[s]).wait()
        pltpu.make_async_copy(dst_hbm.at[pl.ds(0, C)], dstb[bank][s],
                              semL.at[s]).wait()

    def issue_g(s):
        pltpu.async_copy(xs_sh.at[srcb[s]], xgb[s], semG.at[s])

    def drain_g(s):
        pltpu.make_async_copy(xs_sh.at[srcb[s]], xgb[s], semG.at[s]).wait()

    def mul(s):
        xg = xgb[s]
        w = wb[s]

        def mul_body(i, _):
            sl = pl.ds(i * 16, 16)
            xg[sl] = xg[sl] * w[sl]
            return 0
        lax.fori_loop(0, CV, mul_body, 0)

    def issue_s(s, bank):
        pltpu.async_copy(xgb[s], acc_sh.at[dstb[bank][s]], semS.at[s],
                         add=True)

    def drain_s(s, bank):
        pltpu.make_async_copy(xgb[s], acc_sh.at[dstb[bank][s]],
                              semS.at[s]).wait()

    def edge_pass():
        for c in range(NSLOT):
            issue_l(c, c, 0)

        def loop_body(j, _):
            c8 = 2 * NSLOT * j
            for bank in (0, 1):
                for s in range(NSLOT):
                    c = c8 + bank * NSLOT + s
                    @pl.when(c >= NSLOT)
                    def _():
                        drain_s(s, 1 - bank)
                    drain_l(s, bank)
                    issue_g(s)
                for s in range(NSLOT):
                    c = c8 + bank * NSLOT + s
                    drain_g(s)
                    mul(s)
                    issue_s(s, bank)
                    cn = c + NSLOT

                    @pl.when(cn < NCHUNK)
                    def _():
                        issue_l(cn, s, 1 - bank)
            return 0
        lax.fori_loop(0, MAIN // (2 * NSLOT), loop_body, 0)

        for s in range(REM):
            drain_s(s, 1)
            drain_l(s, 0)
            issue_g(s)
        for s in range(REM):
            drain_g(s)
            mul(s)
            issue_s(s, 0)
        for s in range(REM):
            drain_s(s, 0)
        for s in range(REM, NSLOT):
            drain_s(s, 1)

    def stage_and_rezero_acc():
        pltpu.sync_copy(acc_sh.at[psl], tmp_v)
        pltpu.sync_copy(part_hbm.at[psl], tmp2_v)
        pltpu.sync_copy(zer_v, acc_sh.at[psl])
        pltpu.sync_copy(zer_v, acc_sh.at[xsl])

    def publish_xs():
        pltpu.sync_copy(ts_v, xs_sh.at[psl])
        pltpu.sync_copy(ts_v, xsh_hbm.at[psl])
        cross_barrier()
        pltpu.sync_copy(xsh_hbm.at[xsl], xs_sh.at[xsl])
        plsc.subcore_barrier()

    def zero_body(i, _):
        sl = pl.ds(i * 16, 16)
        zer_v[sl] = jnp.zeros((16,), jnp.float32)
        ts_v[sl] = jnp.full((16,), 1.0, jnp.float32)
        return 0
    lax.fori_loop(0, VECS, zero_body, 0)

    pltpu.sync_copy(mu_hbm, mu_v)
    pltpu.sync_copy(yp_hbm.at[psl], yp_v)
    pltpu.sync_copy(zer_v, acc_sh.at[psl])
    pltpu.sync_copy(zer_v, acc_sh.at[xsl])
    pltpu.sync_copy(ts_v, xs_sh.at[psl])
    pltpu.sync_copy(ts_v, xs_sh.at[xsl])
    plsc.subcore_barrier()
    edge_pass()
    plsc.subcore_barrier()
    pltpu.sync_copy(acc_sh.at[xsl], part_hbm.at[xsl])
    cross_barrier()
    stage_and_rezero_acc()

    def init_body(i, _):
        sl = pl.ds(i * 16, 16)
        d = tmp_v[sl] + tmp2_v[sl] + 1e-6
        y = lax.bitcast_convert_type(d, jnp.int32)
        y = 0x5F3759DF - lax.shift_right_arithmetic(y, 1)
        z = lax.bitcast_convert_type(y, jnp.float32)
        z = z * (1.5 - 0.5 * d * z * z)
        z = z * (1.5 - 0.5 * d * z * z)
        z = z * (1.5 - 0.5 * d * z * z)
        dinv_v[sl] = z
        m = jnp.maximum(yp_v[sl], 0.0)
        e = jnp.exp(-2.0 * m)
        t1_v[sl] = (1.0 - e) / (1.0 + e)
        return 0
    lax.fori_loop(0, VECS, init_body, 0)

    mu0 = mu_v[pl.ds(0, 16)]

    def iter_body(it, _):
        def pre_body(i, _):
            sl = pl.ds(i * 16, 16)
            x = t1_v[sl]
            out_v[sl] = mu0 * x
            t0_v[sl] = jnp.zeros((16,), jnp.float32)
            ts_v[sl] = 0.5 * dinv_v[sl] * x
            return 0
        lax.fori_loop(0, VECS, pre_body, 0)
        publish_xs()

        def k_step(k, publish):
            edge_pass()
            plsc.subcore_barrier()
            pltpu.sync_copy(acc_sh.at[xsl], part_hbm.at[xsl])
            cross_barrier()
            stage_and_rezero_acc()
            muk = mu_v[pl.ds(k * 16, 16)]

            def el_body(i, _):
                sl = pl.ds(i * 16, 16)
                dv = dinv_v[sl]
                lapres = -(dv * (tmp_v[sl] + tmp2_v[sl]))
                t2 = 2.0 * lapres - t0_v[sl]
                out_v[sl] = out_v[sl] + muk * t2
                t0_v[sl] = t1_v[sl]
                t1_v[sl] = t2
                ts_v[sl] = dv * t2
                return 0
            lax.fori_loop(0, VECS, el_body, 0)
            if publish:
                publish_xs()

        def k_body(k, _):
            k_step(k, True)
            return 0
        lax.fori_loop(1, K, k_body, 0)

        def post_body(i, _):
            sl = pl.ds(i * 16, 16)
            v = out_v[sl] + yp_v[sl]
            m = jnp.maximum(v, 0.0)
            e = jnp.exp(-2.0 * m)
            t1_v[sl] = (1.0 - e) / (1.0 + e)
            return 0
        lax.fori_loop(0, VECS, post_body, 0)
        return 0
    lax.fori_loop(0, N_ITERS - 1, iter_body, 0)

    pltpu.sync_copy(t1_v, xout_hbm.at[psl])


_sc_call = pl.kernel(
    _sc_body,
    out_type=(
        jax.ShapeDtypeStruct((NPXP,), jnp.float32),
        jax.ShapeDtypeStruct((NPXP,), jnp.float32),
        jax.ShapeDtypeStruct((NPXP,), jnp.float32),
    ),
    mesh=plsc.VectorSubcoreMesh(core_axis_name="c", subcore_axis_name="s",
                                num_cores=NC),
    scratch_types=[
        pltpu.VMEM_SHARED((NPXP,), jnp.float32),
        pltpu.VMEM_SHARED((NPXP,), jnp.float32),
        pltpu.VMEM((SLICE,), jnp.float32),
        pltpu.VMEM((SLICE,), jnp.float32),
        pltpu.VMEM((SLICE,), jnp.float32),
        pltpu.VMEM((SLICE,), jnp.float32),
        pltpu.VMEM((SLICE,), jnp.float32),
        pltpu.VMEM((SLICE,), jnp.float32),
        pltpu.VMEM((SLICE,), jnp.float32),
        pltpu.VMEM((SLICE,), jnp.float32),
        pltpu.VMEM((SLICE,), jnp.float32),
        *([pltpu.VMEM((C,), jnp.int32)] * NSLOT),
        *([pltpu.VMEM((C,), jnp.float32)] * NSLOT),
        *([pltpu.VMEM((C,), jnp.float32)] * NSLOT),
        *([pltpu.VMEM((C,), jnp.int32)] * NSLOT),
        *([pltpu.VMEM((C,), jnp.int32)] * NSLOT),
        pltpu.VMEM((K * 16,), jnp.float32),
        pltpu.SemaphoreType.DMA((NSLOT,)),
        pltpu.SemaphoreType.DMA((NSLOT,)),
        pltpu.SemaphoreType.DMA((NSLOT,)),
        pltpu.SemaphoreType.REGULAR,
    ],
)


@jax.jit
def kernel(S, edge_index, edge_weight, tau, D, mu):
    src = edge_index[0]
    dst = edge_index[1]
    Dp = jnp.pad(D, ((0, 0), (0, NPXP - NPX)))
    taup = jnp.pad(tau, (0, NPXP - NPX)).reshape(1, NPXP)
    mu_rep = jnp.repeat(mu, 16)
    yproj = _yproj(S, Dp, taup).reshape(NPXP)
    xout, _, _ = _sc_call(src, dst, edge_weight, yproj, mu_rep)
    return xout[:NPX]


# speedup vs baseline: 1.1908x; 1.1908x over previous
import jax
import jax.numpy as jnp
from jax import lax
from jax.experimental import pallas as pl
from jax.experimental.pallas import tpu as pltpu
from jax.experimental.pallas import tpu_sc as plsc

NPX = 100000
NCH = 64
E = 1600000
K = 10
N_ITERS = 5

NC = 2
NT = 16
NW = NC * NT
SLICE = 3200
NPXP = NW * SLICE
HALF = NPXP // NC
VECS = SLICE // 16
EPT = E // NW
C = 2000
NCHUNK = EPT // C
CV = C // 16
TCBLK = 1024
NSLOT = 4
MAIN = (NCHUNK // (2 * NSLOT)) * 2 * NSLOT
REM = NCHUNK - MAIN
assert 0 < REM <= NSLOT and MAIN >= 2 * NSLOT and C % 16 == 0
assert (MAIN // NSLOT) % 2 == 0


def _yproj_body(s_ref, d_ref, tau_ref, o_ref):
    s = s_ref[...]
    d = d_ref[...]
    sd = jnp.dot(s, d, preferred_element_type=jnp.float32)
    o_ref[...] = tau_ref[...] * jnp.sum(d * sd, axis=0, keepdims=True)


def _yproj(S, Dp, taup):
    return pl.pallas_call(
        _yproj_body,
        grid=(NPXP // TCBLK,),
        in_specs=[
            pl.BlockSpec((NCH, NCH), lambda i: (0, 0)),
            pl.BlockSpec((NCH, TCBLK), lambda i: (0, i)),
            pl.BlockSpec((1, TCBLK), lambda i: (0, i)),
        ],
        out_specs=pl.BlockSpec((1, TCBLK), lambda i: (0, i)),
        out_shape=jax.ShapeDtypeStruct((1, NPXP), jnp.float32),
    )(S, Dp, taup)


def _sc_body(src_hbm, dst_hbm, w_hbm, yp_hbm, mu_hbm,
             xout_hbm, part_hbm, xsh_hbm,
             xs_sh, acc_sh,
             t0_v, t1_v, out_v, yp_v, dinv_v, tmp_v, tmp2_v, ts_v, zer_v,
             src0, src1, src2, src3,
             w0, w1, w2, w3,
             xg0, xg1, xg2, xg3,
             da0, da1, da2, da3,
             db0, db1, db2, db3,
             mu_v, semL, semG, semS, gsem):
    cid = lax.axis_index("c")
    sid = lax.axis_index("s")
    wid = cid * NT + sid
    gown = cid * HALF + sid * SLICE
    fsl = (1 - cid) * HALF + sid * SLICE
    ebase = wid * EPT
    psl = pl.ds(gown, SLICE)
    xsl = pl.ds(fsl, SLICE)
    srcb = (src0, src1, src2, src3)
    wb = (w0, w1, w2, w3)
    xgb = (xg0, xg1, xg2, xg3)
    dstb = ((da0, da1, da2, da3), (db0, db1, db2, db3))

    def cross_barrier():
        plsc.subcore_barrier()

        @pl.when(sid == 0)
        def _():
            pl.semaphore_signal(gsem, 1, device_id={"c": 1 - cid, "s": 0})
            pl.semaphore_wait(gsem, 1)
        plsc.subcore_barrier()

    def issue_l(c, s, bank):
        eb = ebase + c * C
        pltpu.async_copy(src_hbm.at[pl.ds(eb, C)], srcb[s], semL.at[s])
        pltpu.async_copy(w_hbm.at[pl.ds(eb, C)], wb[s], semL.at[s])
        pltpu.async_copy(dst_hbm.at[pl.ds(eb, C)], dstb[bank][s], semL.at[s])

    def drain_l(s, bank):
        pltpu.make_async_copy(src_hbm.at[pl.ds(0, C)], srcb[s],
                              semL.at[s]).wait()
        pltpu.make_async_copy(w_hbm.at[pl.ds(0, C)], wb[s],
                              semL.at[s]).wait()
        pltpu.make_async_copy(dst_hbm.at[pl.ds(0, C)], dstb[bank][s],
                              semL.at[s]).wait()

    def issue_g(s):
        pltpu.async_copy(xs_sh.at[srcb[s]], xgb[s], semG.at[s])

    def drain_g(s):
        pltpu.make_async_copy(xs_sh.at[srcb[s]], xgb[s], semG.at[s]).wait()

    MULU = 5
    assert CV % MULU == 0

    def mul(s):
        xg = xgb[s]
        w = wb[s]

        def mul_body(i, _):
            for u in range(MULU):
                sl = pl.ds((i * MULU + u) * 16, 16)
                xg[sl] = xg[sl] * w[sl]
            return 0
        lax.fori_loop(0, CV // MULU, mul_body, 0)

    def issue_s(s, bank):
        pltpu.async_copy(xgb[s], acc_sh.at[dstb[bank][s]], semS.at[s],
                         add=True)

    def drain_s(s, bank):
        pltpu.make_async_copy(xgb[s], acc_sh.at[dstb[bank][s]],
                              semS.at[s]).wait()

    def edge_pass():
        for c in range(NSLOT):
            issue_l(c, c, 0)

        def loop_body(j, _):
            c8 = 2 * NSLOT * j
            for bank in (0, 1):
                for s in range(NSLOT):
                    c = c8 + bank * NSLOT + s
                    @pl.when(c >= NSLOT)
                    def _():
                        drain_s(s, 1 - bank)
                    drain_l(s, bank)
                    issue_g(s)
                for s in range(NSLOT):
                    c = c8 + bank * NSLOT + s
                    drain_g(s)
                    mul(s)
                    issue_s(s, bank)
                    cn = c + NSLOT

                    @pl.when(cn < NCHUNK)
                    def _():
                        issue_l(cn, s, 1 - bank)
            return 0
        lax.fori_loop(0, MAIN // (2 * NSLOT), loop_body, 0)

        for s in range(REM):
            drain_s(s, 1)
            drain_l(s, 0)
            issue_g(s)
        for s in range(REM):
            drain_g(s)
            mul(s)
            issue_s(s, 0)
        for s in range(REM):
            drain_s(s, 0)
        for s in range(REM, NSLOT):
            drain_s(s, 1)

    def stage_and_rezero_acc():
        pltpu.sync_copy(acc_sh.at[psl], tmp_v)
        pltpu.sync_copy(part_hbm.at[psl], tmp2_v)
        pltpu.sync_copy(zer_v, acc_sh.at[psl])
        pltpu.sync_copy(zer_v, acc_sh.at[xsl])

    def publish_xs():
        pltpu.sync_copy(ts_v, xs_sh.at[psl])
        pltpu.sync_copy(ts_v, xsh_hbm.at[psl])
        cross_barrier()
        pltpu.sync_copy(xsh_hbm.at[xsl], xs_sh.at[xsl])
        plsc.subcore_barrier()

    def zero_body(i, _):
        sl = pl.ds(i * 16, 16)
        zer_v[sl] = jnp.zeros((16,), jnp.float32)
        ts_v[sl] = jnp.full((16,), 1.0, jnp.float32)
        return 0
    lax.fori_loop(0, VECS, zero_body, 0)

    pltpu.sync_copy(mu_hbm, mu_v)
    pltpu.sync_copy(yp_hbm.at[psl], yp_v)
    pltpu.sync_copy(zer_v, acc_sh.at[psl])
    pltpu.sync_copy(zer_v, acc_sh.at[xsl])
    pltpu.sync_copy(ts_v, xs_sh.at[psl])
    pltpu.sync_copy(ts_v, xs_sh.at[xsl])
    plsc.subcore_barrier()
    edge_pass()
    plsc.subcore_barrier()
    pltpu.sync_copy(acc_sh.at[xsl], part_hbm.at[xsl])
    cross_barrier()
    stage_and_rezero_acc()

    def init_body(i, _):
        sl = pl.ds(i * 16, 16)
        d = tmp_v[sl] + tmp2_v[sl] + 1e-6
        y = lax.bitcast_convert_type(d, jnp.int32)
        y = 0x5F3759DF - lax.shift_right_arithmetic(y, 1)
        z = lax.bitcast_convert_type(y, jnp.float32)
        z = z * (1.5 - 0.5 * d * z * z)
        z = z * (1.5 - 0.5 * d * z * z)
        z = z * (1.5 - 0.5 * d * z * z)
        dinv_v[sl] = z
        m = jnp.maximum(yp_v[sl], 0.0)
        e = jnp.exp(-2.0 * m)
        t1_v[sl] = (1.0 - e) / (1.0 + e)
        return 0
    lax.fori_loop(0, VECS, init_body, 0)

    mu0 = mu_v[pl.ds(0, 16)]

    def iter_body(it, _):
        def pre_body(i, _):
            sl = pl.ds(i * 16, 16)
            x = t1_v[sl]
            out_v[sl] = mu0 * x
            t0_v[sl] = jnp.zeros((16,), jnp.float32)
            ts_v[sl] = 0.5 * dinv_v[sl] * x
            return 0
        lax.fori_loop(0, VECS, pre_body, 0)
        publish_xs()

        def k_step(k, publish):
            edge_pass()
            plsc.subcore_barrier()
            pltpu.sync_copy(acc_sh.at[xsl], part_hbm.at[xsl])
            cross_barrier()
            stage_and_rezero_acc()
            muk = mu_v[pl.ds(k * 16, 16)]

            def el_body(i, _):
                sl = pl.ds(i * 16, 16)
                dv = dinv_v[sl]
                lapres = -(dv * (tmp_v[sl] + tmp2_v[sl]))
                t2 = 2.0 * lapres - t0_v[sl]
                out_v[sl] = out_v[sl] + muk * t2
                t0_v[sl] = t1_v[sl]
                t1_v[sl] = t2
                ts_v[sl] = dv * t2
                return 0
            lax.fori_loop(0, VECS, el_body, 0)
            if publish:
                publish_xs()

        def k_body(k, _):
            k_step(k, True)
            return 0
        lax.fori_loop(1, K, k_body, 0)

        def post_body(i, _):
            sl = pl.ds(i * 16, 16)
            v = out_v[sl] + yp_v[sl]
            m = jnp.maximum(v, 0.0)
            e = jnp.exp(-2.0 * m)
            t1_v[sl] = (1.0 - e) / (1.0 + e)
            return 0
        lax.fori_loop(0, VECS, post_body, 0)
        return 0
    lax.fori_loop(0, N_ITERS - 1, iter_body, 0)

    pltpu.sync_copy(t1_v, xout_hbm.at[psl])


_sc_call = pl.kernel(
    _sc_body,
    out_type=(
        jax.ShapeDtypeStruct((NPXP,), jnp.float32),
        jax.ShapeDtypeStruct((NPXP,), jnp.float32),
        jax.ShapeDtypeStruct((NPXP,), jnp.float32),
    ),
    mesh=plsc.VectorSubcoreMesh(core_axis_name="c", subcore_axis_name="s",
                                num_cores=NC),
    scratch_types=[
        pltpu.VMEM_SHARED((NPXP,), jnp.float32),
        pltpu.VMEM_SHARED((NPXP,), jnp.float32),
        pltpu.VMEM((SLICE,), jnp.float32),
        pltpu.VMEM((SLICE,), jnp.float32),
        pltpu.VMEM((SLICE,), jnp.float32),
        pltpu.VMEM((SLICE,), jnp.float32),
        pltpu.VMEM((SLICE,), jnp.float32),
        pltpu.VMEM((SLICE,), jnp.float32),
        pltpu.VMEM((SLICE,), jnp.float32),
        pltpu.VMEM((SLICE,), jnp.float32),
        pltpu.VMEM((SLICE,), jnp.float32),
        *([pltpu.VMEM((C,), jnp.int32)] * NSLOT),
        *([pltpu.VMEM((C,), jnp.float32)] * NSLOT),
        *([pltpu.VMEM((C,), jnp.float32)] * NSLOT),
        *([pltpu.VMEM((C,), jnp.int32)] * NSLOT),
        *([pltpu.VMEM((C,), jnp.int32)] * NSLOT),
        pltpu.VMEM((K * 16,), jnp.float32),
        pltpu.SemaphoreType.DMA((NSLOT,)),
        pltpu.SemaphoreType.DMA((NSLOT,)),
        pltpu.SemaphoreType.DMA((NSLOT,)),
        pltpu.SemaphoreType.REGULAR,
    ],
)


@jax.jit
def kernel(S, edge_index, edge_weight, tau, D, mu):
    src = edge_index[0]
    dst = edge_index[1]
    Dp = jnp.pad(D, ((0, 0), (0, NPXP - NPX)))
    taup = jnp.pad(tau, (0, NPXP - NPX)).reshape(1, NPXP)
    mu_rep = jnp.repeat(mu, 16)
    yproj = _yproj(S, Dp, taup).reshape(NPXP)
    xout, _, _ = _sc_call(src, dst, edge_weight, yproj, mu_rep)
    return xout[:NPX]
